# Initial kernel scaffold; baseline (speedup 1.0000x reference)
import jax
import jax.numpy as jnp
from jax.experimental import pallas as pl


def kernel(entity_embds, rel_embds):
    raise NotImplementedError("write your pallas kernel here")


# speedup vs baseline: 1.0491x; 1.0491x over previous
import functools

import jax
import jax.numpy as jnp
from jax.experimental import pallas as pl


def _norm_body(x_ref, o_ref, *, block_rows, total_rows):
    i = pl.program_id(0)
    x = x_ref[...]
    ssq = jnp.sum(x * x, axis=1, keepdims=True)
    inv = jax.lax.rsqrt(ssq)
    row = i * block_rows + jax.lax.broadcasted_iota(jnp.int32, (block_rows, 1), 0)
    scale = jnp.where(row == total_rows - 1, 1.0, inv)
    o_ref[...] = x * scale


def kernel(entity_embds, rel_embds):
    n, d = entity_embds.shape
    block = 8000
    out = pl.pallas_call(
        functools.partial(_norm_body, block_rows=block, total_rows=n),
        grid=(n // block,),
        in_specs=[pl.BlockSpec((block, d), lambda i: (i, 0))],
        out_specs=pl.BlockSpec((block, d), lambda i: (i, 0)),
        out_shape=jax.ShapeDtypeStruct((n, d), entity_embds.dtype),
    )(entity_embds)
    return (out, rel_embds)


# speedup vs baseline: 1.0560x; 1.0066x over previous
import functools

import jax
import jax.numpy as jnp
from jax.experimental import pallas as pl


def _norm_body(x_ref, o_ref, *, block_rows, total_rows):
    i = pl.program_id(0)
    x = x_ref[...]
    ssq = jnp.sum(x * x, axis=1, keepdims=True)
    inv = jax.lax.rsqrt(ssq)
    row = i * block_rows + jax.lax.broadcasted_iota(jnp.int32, (block_rows, 1), 0)
    scale = jnp.where(row == total_rows - 1, 1.0, inv)
    o_ref[...] = x * scale


def kernel(entity_embds, rel_embds):
    n, d = entity_embds.shape
    block = 20000
    out = pl.pallas_call(
        functools.partial(_norm_body, block_rows=block, total_rows=n),
        grid=(n // block,),
        in_specs=[pl.BlockSpec((block, d), lambda i: (i, 0))],
        out_specs=pl.BlockSpec((block, d), lambda i: (i, 0)),
        out_shape=jax.ShapeDtypeStruct((n, d), entity_embds.dtype),
    )(entity_embds)
    return (out, rel_embds)
